# baseline (speedup 1.0000x reference)
import functools

import jax
import jax.numpy as jnp
from jax import lax
from jax.experimental import pallas as pl
from jax.experimental.pallas import tpu as pltpu
from jax.experimental.pallas import tpu_sc as plsc

B = 16384
D = 32
WIDE = 128
NUMF = 10
HID = 128
NC, NS = 2, 16
NW = NC * NS
BPW = B // NW
CHUNK = 128
NCHUNK = BPW // CHUNK
RC = D // 8

NU, NM, ND = 1000000, 100000, 10000
TPR_U, TPR_M, TPR_D = (NU + 127) // 128, (NM + 127) // 128, (ND + 127) // 128
NTU, NTM, NTD = NU // 128, NM // 128, ND // 128
LIM_U, LIM_M, LIM_D = NTU * 128, NTM * 128, NTD * 128
TAIL_U, TAIL_M, TAIL_D = NU - LIM_U, NM - LIM_M, ND - LIM_D


def _mesh():
    return plsc.VectorSubcoreMesh(
        core_axis_name="c", subcore_axis_name="s", num_cores=NC, num_subcores=NS)


KT = 8


def _round(src, dst, rc, t0, tpr, bufA, bufB, sems, osem, p, fire_t):
    cp = pltpu.async_copy(
        src.at[pl.ds(pl.multiple_of(rc * 8, 8), 8),
               pl.ds(pl.multiple_of(fire_t * 128, 128), KT * 128)],
        bufA.at[p ^ 1], sems[p ^ 1])
    for t2 in range(KT):
        for s in range(8):
            for l0 in range(0, 128, 16):
                bufB[p, t2 * 8 + s, pl.ds(l0, 16)] = (
                    bufA[p, s, pl.ds(t2 * 128 + l0, 16)])
    ocp = pltpu.async_copy(
        bufB.at[p],
        dst.at[pl.ds(pl.multiple_of((rc * tpr + t0) * 8, 8), KT * 8)],
        osem)
    return cp, ocp


def _dump_tiles(wid, src, dst, tpr, nt, bufA, bufB, sems, osem):
    rc = wid >> 3
    sub = wid & 7
    per = (nt + 7) // 8
    nr = (per + KT - 1) // KT

    def strip_start(r):
        return jnp.minimum(sub * per + r * KT, nt - KT)

    pltpu.async_copy(
        src.at[pl.ds(pl.multiple_of(rc * 8, 8), 8),
               pl.ds(pl.multiple_of(strip_start(0) * 128, 128), KT * 128)],
        bufA.at[1], sems[1]).wait()

    def pair(r2, carry):
        r0 = r2 * 2
        cp0, ocp0 = _round(src, dst, rc, strip_start(r0), tpr, bufA, bufB,
                           sems, osem, 1, strip_start(r0 + 1))
        cp0.wait()
        cp1, ocp1 = _round(src, dst, rc, strip_start(r0 + 1), tpr, bufA,
                           bufB, sems, osem, 0, strip_start(r0 + 2))
        cp1.wait()
        ocp0.wait()
        ocp1.wait()
        return carry

    npairs = (nr + 1) // 2
    lax.fori_loop(0, npairs, pair, 0)


def _sc_dump(uT, mT, dT):
    out_types = (
        jax.ShapeDtypeStruct((RC * TPR_U * 8, 128), jnp.float32),
        jax.ShapeDtypeStruct((RC * TPR_M * 8, 128), jnp.float32),
        jax.ShapeDtypeStruct((RC * TPR_D * 8, 128), jnp.float32),
    )

    @functools.partial(
        pl.kernel,
        out_type=out_types,
        mesh=_mesh(),
        scratch_types=[
            pltpu.VMEM((2, 8, KT * 128), jnp.float32),
            pltpu.VMEM((2, KT * 8, 128), jnp.float32),
            pltpu.SemaphoreType.DMA,
            pltpu.SemaphoreType.DMA,
            pltpu.SemaphoreType.DMA,
        ],
    )
    def k(uT_hbm, mT_hbm, dT_hbm, xu, xm, xd, bufA, bufB, sem0, sem1, osem):
        wid = lax.axis_index("s") * NC + lax.axis_index("c")
        sems = (sem0, sem1)
        _dump_tiles(wid, uT_hbm, xu, TPR_U, NTU, bufA, bufB, sems, osem)
        _dump_tiles(wid, mT_hbm, xm, TPR_M, NTM, bufA, bufB, sems, osem)
        _dump_tiles(wid, dT_hbm, xd, TPR_D, NTD, bufA, bufB, sems, osem)

    return k(uT, mT, dT)


def _sc_gather(uids, mids, dids, xfu, xfm, xfd):
    out_shape = jax.ShapeDtypeStruct((B, WIDE), jnp.float32)

    @functools.partial(
        pl.kernel,
        out_type=(out_shape, out_shape, out_shape),
        mesh=_mesh(),
        scratch_types=[
            pltpu.VMEM((3, NCHUNK, CHUNK), jnp.int32),
            pltpu.VMEM((3, 32, CHUNK), jnp.int32),
            pltpu.VMEM((3, 32, CHUNK), jnp.float32),
            pltpu.VMEM((3, CHUNK, D), jnp.float32),
            pltpu.SemaphoreType.DMA,
        ],
        compiler_params=pltpu.CompilerParams(
            use_tc_tiling_on_sc=False, needs_layout_passes=False),
    )
    def k(uids_hbm, mids_hbm, dids_hbm, xfu_hbm, xfm_hbm, xfd_hbm,
          out_u, out_m, out_d, idx, eidx, got, rows, sem):
        wid = lax.axis_index("s") * NC + lax.axis_index("c")
        base = wid * BPW
        srcs = (xfu_hbm, xfm_hbm, xfd_hbm)
        ids_hbms = (uids_hbm, mids_hbm, dids_hbm)
        outs = (out_u, out_m, out_d)
        tprs = (TPR_U, TPR_M, TPR_D)
        lims = (LIM_U, LIM_M, LIM_D)
        for t in range(3):
            for j in range(NCHUNK):
                pltpu.sync_copy(
                    ids_hbms[t].at[pl.ds(base + j * CHUNK, CHUNK)],
                    idx.at[t, j])

        def chunk_body(j, carry):
            for t in range(3):
                for g in range(CHUNK // 16):
                    idv = idx[t, j, pl.ds(g * 16, 16)]
                    idc = jnp.minimum(idv, lims[t] - 1)
                    a = ((idc >> 7) << 10) + (idc & 127)
                    for c in range(D):
                        kc = ((c >> 3) * tprs[t]) * 1024 + (c & 7) * 128
                        eidx[t, c, pl.ds(g * 16, 16)] = a + kc
            cps = []
            for t in range(3):
                for c in range(D):
                    cps.append(pltpu.async_copy(
                        srcs[t].at[eidx.at[t, c]], got.at[t, c], sem))
            for cp in cps:
                cp.wait()
            for t in range(3):
                for g in range(CHUNK // 16):
                    ridx = lax.iota(jnp.int32, 16) + g * 16
                    for c in range(D):
                        plsc.store_scatter(
                            rows.at[t],
                            [ridx, jnp.full((16,), c, jnp.int32)],
                            got[t, c, pl.ds(g * 16, 16)])
            for t in range(3):
                pltpu.sync_copy(
                    rows.at[t],
                    outs[t].at[pl.ds(base + j * CHUNK, CHUNK), pl.ds(0, D)])
            return carry

        lax.fori_loop(0, NCHUNK, chunk_body, 0)

    return k(uids, mids, dids, xfu, xfm, xfd)


BLK = 2048


def _fix_tail(wide_ref, id_ref, tail_ref, lim, ntail):
    e = wide_ref[:, :D]
    ids = id_ref[...]
    off = ids - lim
    oh = (lax.broadcasted_iota(jnp.int32, (BLK, ntail), 1) == off
          ).astype(jnp.float32)
    fix = jnp.dot(oh, tail_ref[...], preferred_element_type=jnp.float32)
    return jnp.where(ids >= lim, 0.0, e) + fix


def _mlp_body(u_ref, m_ref, d_ref, uid_ref, mid_ref, did_ref, nf_ref,
              ut_tail_ref, mt_tail_ref, dt_tail_ref,
              w1a_ref, w1b_ref, b1_ref, w2_ref, b2_ref, o_ref):
    eu = _fix_tail(u_ref, uid_ref, ut_tail_ref, LIM_U, TAIL_U)
    em = _fix_tail(m_ref, mid_ref, mt_tail_ref, LIM_M, TAIL_M)
    ed = _fix_tail(d_ref, did_ref, dt_tail_ref, LIM_D, TAIL_D)
    e = (eu + em + ed) * (1.0 / 3.0)
    h = jnp.dot(e, w1a_ref[...], preferred_element_type=jnp.float32)
    h = h + jnp.dot(nf_ref[...], w1b_ref[...], preferred_element_type=jnp.float32)
    h = jnp.maximum(h + b1_ref[...], 0.0)
    out = jnp.dot(h, w2_ref[...], preferred_element_type=jnp.float32) + b2_ref[...]
    norm = jnp.sqrt(jnp.sum(out * out, axis=1, keepdims=True))
    o_ref[...] = out / jnp.maximum(norm, 1e-12)


def _mlp(u, m, d, uid, mid, did, nf, ut_tail, mt_tail, dt_tail,
         w1a, w1b, b1, w2, b2):
    full = lambda shape: pl.BlockSpec(shape, lambda i: (0, 0))
    blk = lambda minor: pl.BlockSpec((BLK, minor), lambda i: (i, 0))
    return pl.pallas_call(
        _mlp_body,
        grid=(B // BLK,),
        in_specs=[
            blk(WIDE), blk(WIDE), blk(WIDE),
            blk(1), blk(1), blk(1),
            blk(NUMF),
            full((TAIL_U, D)), full((TAIL_M, D)), full((TAIL_D, D)),
            full((D, HID)),
            full((NUMF, HID)),
            full((1, HID)),
            full((HID, HID)),
            full((1, HID)),
        ],
        out_specs=pl.BlockSpec((BLK, HID), lambda i: (i, 0)),
        out_shape=jax.ShapeDtypeStruct((B, HID), jnp.float32),
    )(u, m, d, uid, mid, did, nf, ut_tail, mt_tail, dt_tail,
      w1a, w1b, b1, w2, b2)


def kernel(user_ids, merchant_ids, device_ids, numerical_features,
           user_table, merchant_table, device_table, W1, b1, W2, b2):
    xu, xm, xd = _sc_dump(user_table.T, merchant_table.T, device_table.T)
    u_rows, m_rows, d_rows = _sc_gather(
        user_ids, merchant_ids, device_ids,
        xu.reshape(-1), xm.reshape(-1), xd.reshape(-1))
    out = _mlp(
        u_rows, m_rows, d_rows,
        user_ids.reshape(B, 1), merchant_ids.reshape(B, 1),
        device_ids.reshape(B, 1), numerical_features,
        user_table[LIM_U:], merchant_table[LIM_M:], device_table[LIM_D:],
        W1[:D], W1[D:], b1.reshape(1, HID), W2, b2.reshape(1, HID))
    return out


# speedup vs baseline: 1.0023x; 1.0023x over previous
import functools

import jax
import jax.numpy as jnp
from jax import lax
from jax.experimental import pallas as pl
from jax.experimental.pallas import tpu as pltpu
from jax.experimental.pallas import tpu_sc as plsc

B = 16384
D = 32
WIDE = 128
NUMF = 10
HID = 128
NC, NS = 2, 16
NW = NC * NS
BPW = B // NW
CHUNK = 128
NCHUNK = BPW // CHUNK
RC = D // 8

NU, NM, ND = 1000000, 100000, 10000
TPR_U, TPR_M, TPR_D = (NU + 127) // 128, (NM + 127) // 128, (ND + 127) // 128
NTU, NTM, NTD = NU // 128, NM // 128, ND // 128
LIM_U, LIM_M, LIM_D = NTU * 128, NTM * 128, NTD * 128
TAIL_U, TAIL_M, TAIL_D = NU - LIM_U, NM - LIM_M, ND - LIM_D


def _mesh():
    return plsc.VectorSubcoreMesh(
        core_axis_name="c", subcore_axis_name="s", num_cores=NC, num_subcores=NS)


KT = 16


def _round(src, dst, rc, t0, tpr, bufA, bufB, sems, osem, p, fire_t):
    cp = pltpu.async_copy(
        src.at[pl.ds(pl.multiple_of(rc * 8, 8), 8),
               pl.ds(pl.multiple_of(fire_t * 128, 128), KT * 128)],
        bufA.at[p ^ 1], sems[p ^ 1])
    for t2 in range(KT):
        for s in range(8):
            for l0 in range(0, 128, 16):
                bufB[p, t2 * 8 + s, pl.ds(l0, 16)] = (
                    bufA[p, s, pl.ds(t2 * 128 + l0, 16)])
    ocp = pltpu.async_copy(
        bufB.at[p],
        dst.at[pl.ds(pl.multiple_of((rc * tpr + t0) * 8, 8), KT * 8)],
        osem)
    return cp, ocp


def _dump_tiles(wid, src, dst, tpr, nt, bufA, bufB, sems, osem):
    rc = wid >> 3
    sub = wid & 7
    per = (nt + 7) // 8
    nr = (per + KT - 1) // KT

    def strip_start(r):
        return jnp.minimum(sub * per + r * KT, nt - KT)

    pltpu.async_copy(
        src.at[pl.ds(pl.multiple_of(rc * 8, 8), 8),
               pl.ds(pl.multiple_of(strip_start(0) * 128, 128), KT * 128)],
        bufA.at[1], sems[1]).wait()

    def pair(r2, carry):
        r0 = r2 * 2
        cp0, ocp0 = _round(src, dst, rc, strip_start(r0), tpr, bufA, bufB,
                           sems, osem, 1, strip_start(r0 + 1))
        cp0.wait()
        cp1, ocp1 = _round(src, dst, rc, strip_start(r0 + 1), tpr, bufA,
                           bufB, sems, osem, 0, strip_start(r0 + 2))
        cp1.wait()
        ocp0.wait()
        ocp1.wait()
        return carry

    npairs = (nr + 1) // 2
    lax.fori_loop(0, npairs, pair, 0)


def _sc_dump(uT, mT, dT):
    out_types = (
        jax.ShapeDtypeStruct((RC * TPR_U * 8, 128), jnp.float32),
        jax.ShapeDtypeStruct((RC * TPR_M * 8, 128), jnp.float32),
        jax.ShapeDtypeStruct((RC * TPR_D * 8, 128), jnp.float32),
    )

    @functools.partial(
        pl.kernel,
        out_type=out_types,
        mesh=_mesh(),
        scratch_types=[
            pltpu.VMEM((2, 8, KT * 128), jnp.float32),
            pltpu.VMEM((2, KT * 8, 128), jnp.float32),
            pltpu.SemaphoreType.DMA,
            pltpu.SemaphoreType.DMA,
            pltpu.SemaphoreType.DMA,
        ],
    )
    def k(uT_hbm, mT_hbm, dT_hbm, xu, xm, xd, bufA, bufB, sem0, sem1, osem):
        wid = lax.axis_index("s") * NC + lax.axis_index("c")
        sems = (sem0, sem1)
        _dump_tiles(wid, uT_hbm, xu, TPR_U, NTU, bufA, bufB, sems, osem)
        _dump_tiles(wid, mT_hbm, xm, TPR_M, NTM, bufA, bufB, sems, osem)
        _dump_tiles(wid, dT_hbm, xd, TPR_D, NTD, bufA, bufB, sems, osem)

    return k(uT, mT, dT)


def _sc_gather(uids, mids, dids, xfu, xfm, xfd):
    out_shape = jax.ShapeDtypeStruct((B, WIDE), jnp.float32)

    @functools.partial(
        pl.kernel,
        out_type=(out_shape, out_shape, out_shape),
        mesh=_mesh(),
        scratch_types=[
            pltpu.VMEM((3, NCHUNK, CHUNK), jnp.int32),
            pltpu.VMEM((3, 32, CHUNK), jnp.int32),
            pltpu.VMEM((3, 32, CHUNK), jnp.float32),
            pltpu.VMEM((3, CHUNK, D), jnp.float32),
            pltpu.SemaphoreType.DMA,
        ],
        compiler_params=pltpu.CompilerParams(
            use_tc_tiling_on_sc=False, needs_layout_passes=False),
    )
    def k(uids_hbm, mids_hbm, dids_hbm, xfu_hbm, xfm_hbm, xfd_hbm,
          out_u, out_m, out_d, idx, eidx, got, rows, sem):
        wid = lax.axis_index("s") * NC + lax.axis_index("c")
        base = wid * BPW
        srcs = (xfu_hbm, xfm_hbm, xfd_hbm)
        ids_hbms = (uids_hbm, mids_hbm, dids_hbm)
        outs = (out_u, out_m, out_d)
        tprs = (TPR_U, TPR_M, TPR_D)
        lims = (LIM_U, LIM_M, LIM_D)
        for t in range(3):
            for j in range(NCHUNK):
                pltpu.sync_copy(
                    ids_hbms[t].at[pl.ds(base + j * CHUNK, CHUNK)],
                    idx.at[t, j])

        def chunk_body(j, carry):
            for t in range(3):
                for g in range(CHUNK // 16):
                    idv = idx[t, j, pl.ds(g * 16, 16)]
                    idc = jnp.minimum(idv, lims[t] - 1)
                    a = ((idc >> 7) << 10) + (idc & 127)
                    for c in range(D):
                        kc = ((c >> 3) * tprs[t]) * 1024 + (c & 7) * 128
                        eidx[t, c, pl.ds(g * 16, 16)] = a + kc
            cps = []
            for t in range(3):
                for c in range(D):
                    cps.append(pltpu.async_copy(
                        srcs[t].at[eidx.at[t, c]], got.at[t, c], sem))
            for cp in cps:
                cp.wait()
            for t in range(3):
                for g in range(CHUNK // 16):
                    ridx = lax.iota(jnp.int32, 16) + g * 16
                    for c in range(D):
                        plsc.store_scatter(
                            rows.at[t],
                            [ridx, jnp.full((16,), c, jnp.int32)],
                            got[t, c, pl.ds(g * 16, 16)])
            for t in range(3):
                pltpu.sync_copy(
                    rows.at[t],
                    outs[t].at[pl.ds(base + j * CHUNK, CHUNK), pl.ds(0, D)])
            return carry

        lax.fori_loop(0, NCHUNK, chunk_body, 0)

    return k(uids, mids, dids, xfu, xfm, xfd)


BLK = 2048


def _fix_tail(wide_ref, id_ref, tail_ref, lim, ntail):
    e = wide_ref[:, :D]
    ids = id_ref[...]
    off = ids - lim
    oh = (lax.broadcasted_iota(jnp.int32, (BLK, ntail), 1) == off
          ).astype(jnp.float32)
    fix = jnp.dot(oh, tail_ref[...], preferred_element_type=jnp.float32)
    return jnp.where(ids >= lim, 0.0, e) + fix


def _mlp_body(u_ref, m_ref, d_ref, uid_ref, mid_ref, did_ref, nf_ref,
              ut_tail_ref, mt_tail_ref, dt_tail_ref,
              w1a_ref, w1b_ref, b1_ref, w2_ref, b2_ref, o_ref):
    eu = _fix_tail(u_ref, uid_ref, ut_tail_ref, LIM_U, TAIL_U)
    em = _fix_tail(m_ref, mid_ref, mt_tail_ref, LIM_M, TAIL_M)
    ed = _fix_tail(d_ref, did_ref, dt_tail_ref, LIM_D, TAIL_D)
    e = (eu + em + ed) * (1.0 / 3.0)
    h = jnp.dot(e, w1a_ref[...], preferred_element_type=jnp.float32)
    h = h + jnp.dot(nf_ref[...], w1b_ref[...], preferred_element_type=jnp.float32)
    h = jnp.maximum(h + b1_ref[...], 0.0)
    out = jnp.dot(h, w2_ref[...], preferred_element_type=jnp.float32) + b2_ref[...]
    norm = jnp.sqrt(jnp.sum(out * out, axis=1, keepdims=True))
    o_ref[...] = out / jnp.maximum(norm, 1e-12)


def _mlp(u, m, d, uid, mid, did, nf, ut_tail, mt_tail, dt_tail,
         w1a, w1b, b1, w2, b2):
    full = lambda shape: pl.BlockSpec(shape, lambda i: (0, 0))
    blk = lambda minor: pl.BlockSpec((BLK, minor), lambda i: (i, 0))
    return pl.pallas_call(
        _mlp_body,
        grid=(B // BLK,),
        in_specs=[
            blk(WIDE), blk(WIDE), blk(WIDE),
            blk(1), blk(1), blk(1),
            blk(NUMF),
            full((TAIL_U, D)), full((TAIL_M, D)), full((TAIL_D, D)),
            full((D, HID)),
            full((NUMF, HID)),
            full((1, HID)),
            full((HID, HID)),
            full((1, HID)),
        ],
        out_specs=pl.BlockSpec((BLK, HID), lambda i: (i, 0)),
        out_shape=jax.ShapeDtypeStruct((B, HID), jnp.float32),
    )(u, m, d, uid, mid, did, nf, ut_tail, mt_tail, dt_tail,
      w1a, w1b, b1, w2, b2)


def kernel(user_ids, merchant_ids, device_ids, numerical_features,
           user_table, merchant_table, device_table, W1, b1, W2, b2):
    xu, xm, xd = _sc_dump(user_table.T, merchant_table.T, device_table.T)
    u_rows, m_rows, d_rows = _sc_gather(
        user_ids, merchant_ids, device_ids,
        xu.reshape(-1), xm.reshape(-1), xd.reshape(-1))
    out = _mlp(
        u_rows, m_rows, d_rows,
        user_ids.reshape(B, 1), merchant_ids.reshape(B, 1),
        device_ids.reshape(B, 1), numerical_features,
        user_table[LIM_U:], merchant_table[LIM_M:], device_table[LIM_D:],
        W1[:D], W1[D:], b1.reshape(1, HID), W2, b2.reshape(1, HID))
    return out
